# baseline (speedup 1.0000x reference)
import functools

import jax
import jax.numpy as jnp
from jax import lax
from jax.experimental import pallas as pl
from jax.experimental.pallas import tpu as pltpu
from jax.experimental.pallas import tpu_sc as plsc

N = 8192
D = 4096
L = 16
NC = 2
NS = 16
NW = NC * NS
ROWS_PER_W = N // NW
RB = 8
NB = ROWS_PER_W // RB
NQ = 4
DQ = D // NQ
JC_Q = DQ // L


def _body(x_hbm, perm_hbm, out_hbm, perm_v, in0, in1, q0, q1, q2, q3,
          sin0, sin1, sq0, sq1, sq2, sq3):
    wid = lax.axis_index("s") * NC + lax.axis_index("c")
    row_base = wid * ROWS_PER_W

    ins = [in0, in1]
    outs = [q0, q1, q2, q3]
    sins = [sin0, sin1]
    souts = [sq0, sq1, sq2, sq3]

    pltpu.sync_copy(perm_hbm, perm_v)

    row_splats = [jnp.full((L,), r, dtype=jnp.int32) for r in range(RB)]

    def start_in(b, p):
        pltpu.async_copy(
            x_hbm.at[pl.ds(row_base + b * RB, RB)], ins[p], sins[p])

    def wait_in(b, p):
        pltpu.make_async_copy(
            x_hbm.at[pl.ds(row_base + b * RB, RB)], ins[p], sins[p]).wait()

    def start_out(b, q):
        pltpu.async_copy(
            outs[q],
            out_hbm.at[pl.ds(row_base + b * RB, RB), pl.ds(q * DQ, DQ)],
            souts[q])

    def wait_out(b, q):
        pltpu.make_async_copy(
            outs[q],
            out_hbm.at[pl.ds(row_base + b * RB, RB), pl.ds(q * DQ, DQ)],
            souts[q]).wait()

    def compute_quarter(p, q):
        in_b = ins[p]
        out_b = outs[q]

        @plsc.parallel_loop(0, JC_Q, unroll=4)
        def j_body(j):
            idxs = perm_v[pl.ds((q * JC_Q + j) * L, L)]
            for r in range(RB):
                vals = plsc.load_gather(in_b, [row_splats[r], idxs])
                out_b[r, pl.ds(j * L, L)] = vals

    start_in(0, 0)

    def pair_body(pair, carry):
        for p in range(2):
            b = pair * 2 + p

            @pl.when(b + 1 < NB)
            def _():
                start_in(b + 1, 1 - p)

            wait_in(b, p)
            for q in range(NQ):
                @pl.when(b >= 1)
                def _():
                    wait_out(b - 1, q)

                compute_quarter(p, q)
                start_out(b, q)
        return carry

    lax.fori_loop(0, NB // 2, pair_body, 0)
    for q in range(NQ):
        wait_out(NB - 1, q)


@jax.jit
def _permute_cols(x, perm32):
    mesh = plsc.VectorSubcoreMesh(core_axis_name="c", subcore_axis_name="s")
    kern = functools.partial(
        pl.kernel,
        mesh=mesh,
        out_type=jax.ShapeDtypeStruct((N, D), jnp.float32),
        compiler_params=pltpu.CompilerParams(
            needs_layout_passes=False, skip_device_barrier=True),
        scratch_types=[
            pltpu.VMEM((D,), jnp.int32),
            pltpu.VMEM((RB, D), jnp.float32),
            pltpu.VMEM((RB, D), jnp.float32),
            pltpu.VMEM((RB, DQ), jnp.float32),
            pltpu.VMEM((RB, DQ), jnp.float32),
            pltpu.VMEM((RB, DQ), jnp.float32),
            pltpu.VMEM((RB, DQ), jnp.float32),
            pltpu.SemaphoreType.DMA,
            pltpu.SemaphoreType.DMA,
            pltpu.SemaphoreType.DMA,
            pltpu.SemaphoreType.DMA,
            pltpu.SemaphoreType.DMA,
            pltpu.SemaphoreType.DMA,
        ],
    )(_body)
    return kern(x, perm32)


def kernel(x, permutation):
    perm32 = permutation.astype(jnp.int32)
    return _permute_cols(x, perm32)


# speedup vs baseline: 1.0123x; 1.0123x over previous
import functools

import jax
import jax.numpy as jnp
from jax import lax
from jax.experimental import pallas as pl
from jax.experimental.pallas import tpu as pltpu
from jax.experimental.pallas import tpu_sc as plsc

N = 8192
D = 4096
L = 16
NC = 2
NS = 16
NW = NC * NS
ROWS_PER_W = N // NW
RB = 8
NB = ROWS_PER_W // RB
DH = D // 2
JC_H = DH // L


def _body(x_hbm, perm_hbm, out_hbm, perm_v, in0, in1, out0, out1,
          sin0, sin1, sout0, sout1):
    wid = lax.axis_index("s") * NC + lax.axis_index("c")
    row_base = wid * ROWS_PER_W

    ins = [in0, in1]
    outs = [out0, out1]
    sins = [sin0, sin1]
    souts = [sout0, sout1]

    pltpu.sync_copy(perm_hbm, perm_v)

    row_splats = [jnp.full((L,), r, dtype=jnp.int32) for r in range(RB)]

    def start_in(b, p):
        pltpu.async_copy(
            x_hbm.at[pl.ds(row_base + b * RB, RB)], ins[p], sins[p])

    def wait_in(b, p):
        pltpu.make_async_copy(
            x_hbm.at[pl.ds(row_base + b * RB, RB)], ins[p], sins[p]).wait()

    def start_out(b, h):
        pltpu.async_copy(
            outs[h],
            out_hbm.at[pl.ds(row_base + b * RB, RB), pl.ds(h * DH, DH)],
            souts[h])

    def wait_out(b, h):
        pltpu.make_async_copy(
            outs[h],
            out_hbm.at[pl.ds(row_base + b * RB, RB), pl.ds(h * DH, DH)],
            souts[h]).wait()

    def compute_half(p, h):
        in_b = ins[p]
        out_b = outs[h]

        @plsc.parallel_loop(0, JC_H, unroll=4)
        def j_body(j):
            idxs = perm_v[pl.ds((h * JC_H + j) * L, L)]
            for r in range(RB):
                vals = plsc.load_gather(in_b, [row_splats[r], idxs])
                out_b[r, pl.ds(j * L, L)] = vals

    start_in(0, 0)

    def pair_body(pair, carry):
        for p in range(2):
            b = pair * 2 + p

            @pl.when(b + 1 < NB)
            def _():
                start_in(b + 1, 1 - p)

            wait_in(b, p)
            for h in range(2):
                @pl.when(b >= 1)
                def _():
                    wait_out(b - 1, h)

                compute_half(p, h)
                start_out(b, h)
        return carry

    lax.fori_loop(0, NB // 2, pair_body, 0)
    wait_out(NB - 1, 0)
    wait_out(NB - 1, 1)


@jax.jit
def _permute_cols(x, perm32):
    mesh = plsc.VectorSubcoreMesh(core_axis_name="c", subcore_axis_name="s")
    kern = functools.partial(
        pl.kernel,
        mesh=mesh,
        out_type=jax.ShapeDtypeStruct((N, D), jnp.float32),
        compiler_params=pltpu.CompilerParams(needs_layout_passes=False),
        scratch_types=[
            pltpu.VMEM((D,), jnp.int32),
            pltpu.VMEM((RB, D), jnp.float32),
            pltpu.VMEM((RB, D), jnp.float32),
            pltpu.VMEM((RB, DH), jnp.float32),
            pltpu.VMEM((RB, DH), jnp.float32),
            pltpu.SemaphoreType.DMA,
            pltpu.SemaphoreType.DMA,
            pltpu.SemaphoreType.DMA,
            pltpu.SemaphoreType.DMA,
        ],
    )(_body)
    return kern(x, perm32)


def kernel(x, permutation):
    perm32 = permutation.astype(jnp.int32)
    return _permute_cols(x, perm32)
